# baseline (speedup 1.0000x reference)
import jax
import jax.numpy as jnp
from jax import lax
from jax.experimental import pallas as pl
from jax.experimental.pallas import tpu as pltpu
from jax.experimental.pallas import tpu_sc as plsc

NC = 2
NS = 16
F = 26
FH = F // 2
BPT = 1024
NSTAGE = 8
G = 128


def _body(xidx, table_t, out, idx_v, gout, spmem, gsem0, gsem1,
          osem0, osem1, ssem):
    cid = lax.axis_index("c")
    sid = lax.axis_index("s")
    voc = table_t.shape[1]
    dpc = table_t.shape[0] // NC
    chunk = voc // NSTAGE
    gsems = (gsem0, gsem1)
    osems = (osem0, osem1)
    njg = BPT // G
    d0 = cid * dpc

    @pl.loop(0, F)
    def _idx(f):
        pltpu.sync_copy(xidx.at[f * NS + sid], idx_v.at[f])

    def stage_fire(d):
        @pl.when(sid < NSTAGE)
        def _():
            pltpu.async_copy(
                table_t.at[d0 + d, pl.ds(sid * chunk, chunk)],
                spmem.at[pl.ds(sid * chunk, chunk)], ssem)

    def stage_wait(d):
        @pl.when(sid < NSTAGE)
        def _():
            pltpu.make_async_copy(
                table_t.at[d0 + d, pl.ds(sid * chunk, chunk)],
                spmem.at[pl.ds(sid * chunk, chunk)], ssem).wait()

    def fire_gathers(s):
        @pl.loop(0, FH)
        def _(fh):
            for j in range(njg):
                pltpu.async_copy(
                    spmem.at[idx_v.at[s * FH + fh, pl.ds(j * G, G)]],
                    gout.at[s, fh, pl.ds(j * G, G)], gsems[s])

    def drain_gathers(s):
        @pl.loop(0, FH)
        def _(fh):
            for j in range(njg):
                pltpu.make_async_copy(
                    spmem.at[idx_v.at[s * FH + fh, pl.ds(j * G, G)]],
                    gout.at[s, fh, pl.ds(j * G, G)], gsems[s]).wait()

    def fire_out(s, dg):
        @pl.loop(0, FH)
        def _(fh):
            pltpu.async_copy(
                gout.at[s, fh],
                out.at[s * FH + fh, dg, pl.ds(sid * BPT, BPT)], osems[s])

    def drain_out(s, dg):
        @pl.loop(0, FH)
        def _(fh):
            pltpu.make_async_copy(
                gout.at[s, fh],
                out.at[s * FH + fh, dg, pl.ds(sid * BPT, BPT)],
                osems[s]).wait()

    stage_fire(0)
    stage_wait(0)
    plsc.subcore_barrier()

    @pl.loop(0, dpc)
    def _cols(d):
        dg = d0 + d
        for s in (0, 1):
            @pl.when(d >= 1)
            def _():
                drain_out(s, dg - 1)
            fire_gathers(s)
            drain_gathers(s)
            fire_out(s, dg)
        plsc.subcore_barrier()

        @pl.when(d + 1 < dpc)
        def _():
            stage_fire(d + 1)
            stage_wait(d + 1)
        plsc.subcore_barrier()

    drain_out(0, d0 + dpc - 1)
    drain_out(1, d0 + dpc - 1)


def kernel(x, table):
    B, FF = x.shape
    V, D = table.shape
    xidx = jnp.swapaxes(x, 0, 1).astype(jnp.int32).reshape(FF * NS, B // NS)
    table_t = jnp.swapaxes(table, 0, 1)
    mesh = plsc.VectorSubcoreMesh(core_axis_name="c", subcore_axis_name="s")
    out = pl.kernel(
        _body,
        out_type=jax.ShapeDtypeStruct((FF, D, B), jnp.float32),
        mesh=mesh,
        scratch_types=[
            pltpu.VMEM((F, BPT), jnp.int32),
            pltpu.VMEM((2, FH, BPT), jnp.float32),
            pltpu.VMEM_SHARED((V,), jnp.float32),
            pltpu.SemaphoreType.DMA,
            pltpu.SemaphoreType.DMA,
            pltpu.SemaphoreType.DMA,
            pltpu.SemaphoreType.DMA,
            pltpu.SemaphoreType.DMA,
        ],
        compiler_params=pltpu.CompilerParams(use_tc_tiling_on_sc=False),
    )(xidx, table_t)
    return jnp.transpose(out, (2, 0, 1))


# speedup vs baseline: 3.7533x; 3.7533x over previous
import jax
import jax.numpy as jnp
from jax import lax
from jax.experimental import pallas as pl
from jax.experimental.pallas import tpu as pltpu
from jax.experimental.pallas import tpu_sc as plsc

NC = 2
NS = 16
F = 26
FH = F // 2
BPT = 1024
NSTAGE = 13
SBLK = 601
G = 128
DB = 8
VB = 128


def _body(xidx, tview, out, idx_v, gout, spmem, gsem0, gsem1,
          osem0, osem1, ssem):
    cid = lax.axis_index("c")
    sid = lax.axis_index("s")
    nvb = tview.shape[1]
    dpc = (tview.shape[0] * DB) // NC
    gsems = (gsem0, gsem1)
    osems = (osem0, osem1)
    njg = BPT // G
    d0 = cid * dpc

    @pl.loop(0, F)
    def _idx(f):
        pltpu.sync_copy(xidx.at[f * NS + sid], idx_v.at[f])

    def stage_fire(d):
        dg = d0 + d
        i = dg // DB
        r = dg % DB

        @pl.when(sid < NSTAGE)
        def _():
            @pl.loop(0, SBLK)
            def _st(k):
                j = sid * SBLK + k
                pltpu.async_copy(tview.at[i, j, r],
                                 spmem.at[pl.ds(j * VB, VB)], ssem)

    def stage_wait(d):
        dg = d0 + d
        i = dg // DB
        r = dg % DB

        @pl.when(sid < NSTAGE)
        def _():
            @pl.loop(0, SBLK)
            def _st(k):
                j = sid * SBLK + k
                pltpu.make_async_copy(tview.at[i, j, r],
                                      spmem.at[pl.ds(j * VB, VB)],
                                      ssem).wait()

    def fire_gathers(s):
        @pl.loop(0, FH)
        def _(fh):
            for j in range(njg):
                pltpu.async_copy(
                    spmem.at[idx_v.at[s * FH + fh, pl.ds(j * G, G)]],
                    gout.at[s, fh, j], gsems[s])

    def drain_gathers(s):
        @pl.loop(0, FH)
        def _(fh):
            for j in range(njg):
                pltpu.make_async_copy(
                    spmem.at[idx_v.at[s * FH + fh, pl.ds(j * G, G)]],
                    gout.at[s, fh, j], gsems[s]).wait()

    def fire_out(s, d):
        dg = d0 + d
        i = dg // DB
        r = dg % DB

        @pl.loop(0, FH)
        def _(fh):
            pltpu.async_copy(
                gout.at[s, fh],
                out.at[s * FH + fh, i, pl.ds(sid * DB, DB), r], osems[s])

    def drain_out(s, d):
        dg = d0 + d
        i = dg // DB
        r = dg % DB

        @pl.loop(0, FH)
        def _(fh):
            pltpu.make_async_copy(
                gout.at[s, fh],
                out.at[s * FH + fh, i, pl.ds(sid * DB, DB), r],
                osems[s]).wait()

    stage_fire(0)
    stage_wait(0)
    plsc.subcore_barrier()

    @pl.loop(0, dpc)
    def _cols(d):
        for s in (0, 1):
            @pl.when(d >= 1)
            def _():
                drain_out(s, d - 1)
            fire_gathers(s)
            drain_gathers(s)
            fire_out(s, d)
        plsc.subcore_barrier()

        @pl.when(d + 1 < dpc)
        def _():
            stage_fire(d + 1)
            stage_wait(d + 1)
        plsc.subcore_barrier()

    drain_out(0, dpc - 1)
    drain_out(1, dpc - 1)


def kernel(x, table):
    B, FF = x.shape
    V, D = table.shape
    vpad = (-V) % VB
    nvb = (V + vpad) // VB
    ndb = D // DB
    xidx = jnp.swapaxes(x, 0, 1).astype(jnp.int32).reshape(FF * NS, B // NS)
    tpad = jnp.pad(table, ((0, vpad), (0, 0)))
    tview = (tpad.T.reshape(ndb, DB, nvb, VB).transpose(0, 2, 1, 3))
    mesh = plsc.VectorSubcoreMesh(core_axis_name="c", subcore_axis_name="s")
    out = pl.kernel(
        _body,
        out_type=jax.ShapeDtypeStruct((FF, ndb, B // VB, DB, VB),
                                      jnp.float32),
        mesh=mesh,
        scratch_types=[
            pltpu.VMEM((F, BPT), jnp.int32),
            pltpu.VMEM((2, FH, DB, G), jnp.float32),
            pltpu.VMEM_SHARED((nvb * VB,), jnp.float32),
            pltpu.SemaphoreType.DMA,
            pltpu.SemaphoreType.DMA,
            pltpu.SemaphoreType.DMA,
            pltpu.SemaphoreType.DMA,
            pltpu.SemaphoreType.DMA,
        ],
        compiler_params=pltpu.CompilerParams(use_tc_tiling_on_sc=False),
    )(xidx, tview)
    return (out.transpose(0, 1, 3, 2, 4).reshape(FF, D, B)
            .transpose(2, 0, 1))


# speedup vs baseline: 3.7752x; 1.0058x over previous
import jax
import jax.numpy as jnp
from jax import lax
from jax.experimental import pallas as pl
from jax.experimental.pallas import tpu as pltpu
from jax.experimental.pallas import tpu_sc as plsc

NC = 2
NS = 16
F = 26
FH = F // 2
BPT = 1024
NSTAGE = 13
SBLK = 601
G = 128
DB = 8
VB = 128


def _body(xidx, tview, dummy, out, idx_v, gout, spmem, gsem0, gsem1,
          osem0, osem1, ssem):
    cid = lax.axis_index("c")
    sid = lax.axis_index("s")
    nvb = tview.shape[1]
    dpc = (tview.shape[0] * DB) // NC
    gsems = (gsem0, gsem1)
    osems = (osem0, osem1)
    njg = BPT // G
    d0 = cid * dpc

    @pl.loop(0, F)
    def _idx(f):
        pltpu.sync_copy(xidx.at[f * NS + sid], idx_v.at[f])

    def stage_fire(d):
        dg = d0 + d
        i = dg // DB
        r = dg % DB

        @pl.when(sid < NSTAGE)
        def _():
            @pl.loop(0, SBLK)
            def _st(k):
                j = sid * SBLK + k
                pltpu.async_copy(tview.at[i, j, r],
                                 spmem.at[pl.ds(j * VB, VB)], ssem)

    def stage_wait(d):
        @pl.when(sid < NSTAGE)
        def _():
            pltpu.make_async_copy(
                dummy.at[pl.ds(0, SBLK * VB)],
                spmem.at[pl.ds(sid * SBLK * VB, SBLK * VB)], ssem).wait()

    def fire_gathers(s):
        @pl.loop(0, FH)
        def _(fh):
            for j in range(njg):
                pltpu.async_copy(
                    spmem.at[idx_v.at[s * FH + fh, pl.ds(j * G, G)]],
                    gout.at[s, fh, j], gsems[s])

    def drain_gathers(s):
        pltpu.make_async_copy(tview.at[0, pl.ds(0, FH)], gout.at[s],
                              gsems[s]).wait()

    def fire_out(s, d):
        dg = d0 + d
        i = dg // DB
        r = dg % DB

        @pl.loop(0, FH)
        def _(fh):
            pltpu.async_copy(
                gout.at[s, fh],
                out.at[s * FH + fh, i, pl.ds(sid * DB, DB), r], osems[s])

    def drain_out(s, d):
        pltpu.make_async_copy(tview.at[0, pl.ds(0, FH)], gout.at[s],
                              osems[s]).wait()

    stage_fire(0)
    stage_wait(0)
    plsc.subcore_barrier()

    @pl.loop(0, dpc)
    def _cols(d):
        for s in (0, 1):
            @pl.when(d >= 1)
            def _():
                drain_out(s, d - 1)
            fire_gathers(s)
            drain_gathers(s)
            fire_out(s, d)
        plsc.subcore_barrier()

        @pl.when(d + 1 < dpc)
        def _():
            stage_fire(d + 1)
            stage_wait(d + 1)
        plsc.subcore_barrier()

    drain_out(0, dpc - 1)
    drain_out(1, dpc - 1)


def kernel(x, table):
    B, FF = x.shape
    V, D = table.shape
    vpad = (-V) % VB
    nvb = (V + vpad) // VB
    ndb = D // DB
    xidx = jnp.swapaxes(x, 0, 1).astype(jnp.int32).reshape(FF * NS, B // NS)
    tpad = jnp.pad(table, ((0, vpad), (0, 0)))
    tview = (tpad.T.reshape(ndb, DB, nvb, VB).transpose(0, 2, 1, 3))
    dummy = jnp.zeros((SBLK * VB,), jnp.float32)
    mesh = plsc.VectorSubcoreMesh(core_axis_name="c", subcore_axis_name="s")
    out = pl.kernel(
        _body,
        out_type=jax.ShapeDtypeStruct((FF, ndb, B // VB, DB, VB),
                                      jnp.float32),
        mesh=mesh,
        scratch_types=[
            pltpu.VMEM((F, BPT), jnp.int32),
            pltpu.VMEM((2, FH, DB, G), jnp.float32),
            pltpu.VMEM_SHARED((nvb * VB,), jnp.float32),
            pltpu.SemaphoreType.DMA,
            pltpu.SemaphoreType.DMA,
            pltpu.SemaphoreType.DMA,
            pltpu.SemaphoreType.DMA,
            pltpu.SemaphoreType.DMA,
        ],
        compiler_params=pltpu.CompilerParams(use_tc_tiling_on_sc=False),
    )(xidx, tview, dummy)
    return (out.transpose(0, 1, 3, 2, 4).reshape(FF, D, B)
            .transpose(2, 0, 1))


# speedup vs baseline: 3.7902x; 1.0040x over previous
import jax
import jax.numpy as jnp
from jax import lax
from jax.experimental import pallas as pl
from jax.experimental.pallas import tpu as pltpu
from jax.experimental.pallas import tpu_sc as plsc

NC = 2
NS = 16
F = 26
FH = F // 2
BPT = 1024
SBLK = 488
SREM = 5
G = 128
DB = 8
VB = 128


def _body(xidx, tview, dummy, out, idx_v, gout, spmem, gsem0, gsem1,
          osem0, osem1, ssem):
    cid = lax.axis_index("c")
    sid = lax.axis_index("s")
    nvb = tview.shape[1]
    dpc = (tview.shape[0] * DB) // NC
    gsems = (gsem0, gsem1)
    osems = (osem0, osem1)
    njg = BPT // G
    d0 = cid * dpc

    @pl.loop(0, F)
    def _idx(f):
        pltpu.sync_copy(xidx.at[f * NS + sid], idx_v.at[f])

    def stage_fire(d):
        dg = d0 + d
        i = dg // DB
        r = dg % DB
        base = sid * SBLK

        @pl.loop(0, SBLK, unroll=8)
        def _st(k):
            j = base + k
            pltpu.async_copy(tview.at[i, j, r],
                             spmem.at[pl.ds(j * VB, VB)], ssem)

        @pl.when(sid == 0)
        def _():
            @pl.loop(0, SREM)
            def _st2(k):
                j = NS * SBLK + k
                pltpu.async_copy(tview.at[i, j, r],
                                 spmem.at[pl.ds(j * VB, VB)], ssem)

    def stage_wait(d):
        pltpu.make_async_copy(
            dummy.at[pl.ds(0, SBLK * VB)],
            spmem.at[pl.ds(sid * SBLK * VB, SBLK * VB)], ssem).wait()

        @pl.when(sid == 0)
        def _():
            pltpu.make_async_copy(
                dummy.at[pl.ds(0, SREM * VB)],
                spmem.at[pl.ds(NS * SBLK * VB, SREM * VB)], ssem).wait()

    def fire_gathers(s):
        @pl.loop(0, FH)
        def _(fh):
            for j in range(njg):
                pltpu.async_copy(
                    spmem.at[idx_v.at[s * FH + fh, pl.ds(j * G, G)]],
                    gout.at[s, fh, j], gsems[s])

    def drain_gathers(s):
        pltpu.make_async_copy(tview.at[0, pl.ds(0, FH)], gout.at[s],
                              gsems[s]).wait()

    def fire_out(s, d):
        dg = d0 + d
        i = dg // DB
        r = dg % DB

        @pl.loop(0, FH)
        def _(fh):
            pltpu.async_copy(
                gout.at[s, fh],
                out.at[s * FH + fh, i, pl.ds(sid * DB, DB), r], osems[s])

    def drain_out(s, d):
        pltpu.make_async_copy(tview.at[0, pl.ds(0, FH)], gout.at[s],
                              osems[s]).wait()

    stage_fire(0)
    stage_wait(0)
    plsc.subcore_barrier()

    @pl.loop(0, dpc)
    def _cols(d):
        @pl.when(d >= 1)
        def _():
            drain_out(0, d - 1)
            drain_out(1, d - 1)
        fire_gathers(0)
        fire_gathers(1)
        drain_gathers(0)
        fire_out(0, d)
        drain_gathers(1)
        fire_out(1, d)
        plsc.subcore_barrier()

        @pl.when(d + 1 < dpc)
        def _():
            stage_fire(d + 1)
            stage_wait(d + 1)
        plsc.subcore_barrier()

    drain_out(0, dpc - 1)
    drain_out(1, dpc - 1)


def kernel(x, table):
    B, FF = x.shape
    V, D = table.shape
    vpad = (-V) % VB
    nvb = (V + vpad) // VB
    ndb = D // DB
    xidx = jnp.swapaxes(x, 0, 1).astype(jnp.int32).reshape(FF * NS, B // NS)
    tpad = jnp.pad(table, ((0, vpad), (0, 0)))
    tview = (tpad.T.reshape(ndb, DB, nvb, VB).transpose(0, 2, 1, 3))
    dummy = jnp.zeros((SBLK * VB,), jnp.float32)
    mesh = plsc.VectorSubcoreMesh(core_axis_name="c", subcore_axis_name="s")
    out = pl.kernel(
        _body,
        out_type=jax.ShapeDtypeStruct((FF, ndb, B // VB, DB, VB),
                                      jnp.float32),
        mesh=mesh,
        scratch_types=[
            pltpu.VMEM((F, BPT), jnp.int32),
            pltpu.VMEM((2, FH, DB, G), jnp.float32),
            pltpu.VMEM_SHARED((nvb * VB,), jnp.float32),
            pltpu.SemaphoreType.DMA,
            pltpu.SemaphoreType.DMA,
            pltpu.SemaphoreType.DMA,
            pltpu.SemaphoreType.DMA,
            pltpu.SemaphoreType.DMA,
        ],
        compiler_params=pltpu.CompilerParams(use_tc_tiling_on_sc=False),
    )(xidx, tview, dummy)
    return (out.transpose(0, 1, 3, 2, 4).reshape(FF, D, B)
            .transpose(2, 0, 1))
